# Initial kernel scaffold; baseline (speedup 1.0000x reference)
import jax
import jax.numpy as jnp
from jax.experimental import pallas as pl


def kernel(x, edge_index, W1, b1, att_src1, att_dst1, W2, b2, att_src2, att_dst2):
    raise NotImplementedError("write your pallas kernel here")


# speedup vs baseline: 30.5231x; 30.5231x over previous
import functools

import jax
import jax.numpy as jnp
from jax import lax
from jax.experimental import pallas as pl
from jax.experimental.pallas import tpu as pltpu
from jax.experimental.pallas import tpu_sc as plsc

NC = 2
NS = 16
NW = NC * NS
K = 128
BN = 128


def _cdiv(a, b):
    return (a + b - 1) // b


def _tc_a_body(x_ref, w1_ref, asrc_ref, adst_ref, *out_refs):
    att_ref = out_refs[4]
    hb = jnp.dot(x_ref[...], w1_ref[...],
                 preferred_element_type=jnp.float32,
                 precision=lax.Precision.HIGHEST)
    cols = []
    for h in range(4):
        hh = hb[:, h * 32:(h + 1) * 32]
        out_refs[h][...] = hh
        cols.append(jnp.sum(hh * asrc_ref[h, :][None, :], axis=1))
    for h in range(4):
        hh = hb[:, h * 32:(h + 1) * 32]
        cols.append(jnp.sum(hh * adst_ref[h, :][None, :], axis=1))
    cols.extend([jnp.zeros((BN,), jnp.float32)] * 8)
    att_ref[...] = jnp.stack(cols, axis=1)


def _tc_a(x_p, W1, att_src1, att_dst1, NP):
    grid = NP // BN
    outs = [jax.ShapeDtypeStruct((NP, 32), jnp.float32) for _ in range(4)]
    outs.append(jax.ShapeDtypeStruct((NP, 16), jnp.float32))
    return pl.pallas_call(
        _tc_a_body,
        grid=(grid,),
        in_specs=[
            pl.BlockSpec((BN, 64), lambda i: (i, 0)),
            pl.BlockSpec((64, 128), lambda i: (0, 0)),
            pl.BlockSpec((4, 32), lambda i: (0, 0)),
            pl.BlockSpec((4, 32), lambda i: (0, 0)),
        ],
        out_specs=[pl.BlockSpec((BN, 32), lambda i: (i, 0))
                   for _ in range(4)] +
                  [pl.BlockSpec((BN, 16), lambda i: (i, 0))],
        out_shape=outs,
    )(x_p, W1, att_src1, att_dst1)


def _sc_b(h_heads, att, src_r, dst_r, zer32, zer1, NP, nchunks):
    mesh = plsc.VectorSubcoreMesh(core_axis_name="c", subcore_axis_name="s",
                                  num_cores=NC, num_subcores=NS)
    tecrows = NP // NS

    @functools.partial(
        pl.kernel,
        out_type=[
            jax.ShapeDtypeStruct((NC * 4 * NP, 32), jnp.float32),
            jax.ShapeDtypeStruct((NC * 4 * NP,), jnp.float32),
            jax.ShapeDtypeStruct((NW, nchunks, 4, K), jnp.float32),
        ],
        mesh=mesh,
        scratch_types=[
            pltpu.VMEM((K,), jnp.int32),
            pltpu.VMEM((K,), jnp.int32),
            pltpu.VMEM((K, 16), jnp.float32),
            pltpu.VMEM((K, 16), jnp.float32),
            pltpu.VMEM((4, K), jnp.float32),
            pltpu.VMEM((K, 32), jnp.float32),
            pltpu.VMEM((K,), jnp.float32),
            pltpu.VMEM((NP // NS,), jnp.float32),
            pltpu.VMEM_SHARED((NP, 32), jnp.float32),
            pltpu.VMEM_SHARED((NP,), jnp.float32),
            pltpu.SemaphoreType.DMA,
        ],
        compiler_params=pltpu.CompilerParams(needs_layout_passes=False,
                                             use_tc_tiling_on_sc=False),
    )
    def kern(h0, h1, h2, h3, att_h, src_h, dst_h, z32_h, z1_h,
             aggp, denp, w_r,
             srcb, dstb, abuf_s, abuf_d, wtmp, hbuf, wbuf, zbuf,
             accum, dena, sem):
        c = lax.axis_index("c")
        s = lax.axis_index("s")
        wid = s * NC + c
        hs = (h0, h1, h2, h3)
        row0 = s * tecrows

        @pl.loop(0, nchunks)
        def _pro(ci):
            pltpu.sync_copy(src_h.at[wid, ci], srcb)
            pltpu.sync_copy(dst_h.at[wid, ci], dstb)
            pltpu.async_copy(att_h.at[srcb], abuf_s, sem).wait()
            pltpu.async_copy(att_h.at[dstb], abuf_d, sem).wait()
            for g in range(K // 16):
                rows = lax.iota(jnp.int32, 16) + (g * 16)
                for h in range(4):
                    av = plsc.load_gather(
                        abuf_s, [rows, jnp.full((16,), h, jnp.int32)])
                    bv = plsc.load_gather(
                        abuf_d, [rows, jnp.full((16,), 4 + h, jnp.int32)])
                    e = av + bv
                    e = jnp.where(e > 0, e, 0.2 * e)
                    wtmp[h, pl.ds(g * 16, 16)] = jnp.exp(e)
            pltpu.sync_copy(wtmp, w_r.at[wid, ci])

        for h in range(4):
            pltpu.sync_copy(z32_h.at[pl.ds(row0, tecrows), :],
                            accum.at[pl.ds(row0, tecrows), :])
            pltpu.sync_copy(z1_h.at[pl.ds(row0, tecrows)], zbuf)
            pltpu.sync_copy(zbuf, dena.at[pl.ds(row0, tecrows)])
            plsc.subcore_barrier()

            @pl.loop(0, nchunks)
            def _chunk(ci):
                pltpu.sync_copy(src_h.at[wid, ci], srcb)
                pltpu.sync_copy(dst_h.at[wid, ci], dstb)
                pltpu.sync_copy(w_r.at[wid, ci, h], wbuf)
                pltpu.async_copy(hs[h].at[srcb], hbuf, sem).wait()
                for g in range(K // 16):
                    w = wbuf[pl.ds(g * 16, 16)]
                    for j in range(16):
                        r = g * 16 + j
                        wsp = w[j] * jnp.ones((16,), jnp.float32)
                        hbuf[r, pl.ds(0, 16)] = hbuf[r, pl.ds(0, 16)] * wsp
                        hbuf[r, pl.ds(16, 16)] = hbuf[r, pl.ds(16, 16)] * wsp
                pltpu.sync_copy(hbuf, accum.at[dstb], add=True)
                pltpu.sync_copy(wbuf, dena.at[dstb], add=True)

            plsc.subcore_barrier()
            off = (c * 4 + h) * NP + s * tecrows
            pltpu.sync_copy(accum.at[pl.ds(s * tecrows, tecrows), :],
                            aggp.at[pl.ds(off, tecrows), :])
            pltpu.sync_copy(dena.at[pl.ds(s * tecrows, tecrows)], zbuf)
            pltpu.sync_copy(zbuf, denp.at[pl.ds(off, tecrows)])
            plsc.subcore_barrier()

    return kern(*h_heads, att, src_r, dst_r, zer32, zer1)


def _tc_c_body(aggp_ref, denp_ref, b1_ref, w2_ref, as2_ref, t_ref, NP):
    z = jnp.zeros((BN,), jnp.float32)
    for h in range(4):
        den = denp_ref[0, h, :] + denp_ref[1, h, :]
        agg = aggp_ref[h, :, :] + aggp_ref[4 + h, :, :]
        c0 = h * 32
        h1h = agg / (den[:, None] + 1e-16) + b1_ref[0, c0:c0 + 32][None, :]
        h1h = jnp.maximum(h1h, 0.0)
        z = z + jnp.sum(h1h * w2_ref[0, c0:c0 + 32][None, :], axis=1)
    t_ref[...] = jnp.stack([z * as2_ref[0, 0], z], axis=0)


def _tc_c(aggp, denp, b1, W2, att_src2, NP):
    grid = NP // BN
    aggp4 = aggp.reshape(NC * 4, NP, 32)
    denp4 = denp.reshape(NC, 4, NP)
    body = functools.partial(_tc_c_body, NP=NP)
    return pl.pallas_call(
        body,
        grid=(grid,),
        in_specs=[
            pl.BlockSpec((NC * 4, BN, 32), lambda i: (0, i, 0)),
            pl.BlockSpec((NC, 4, BN), lambda i: (0, 0, i)),
            pl.BlockSpec((1, 128), lambda i: (0, 0)),
            pl.BlockSpec((1, 128), lambda i: (0, 0)),
            pl.BlockSpec((1, 1), lambda i: (0, 0)),
        ],
        out_specs=pl.BlockSpec((2, BN), lambda i: (0, i)),
        out_shape=jax.ShapeDtypeStruct((2, NP), jnp.float32),
    )(aggp4, denp4, b1.reshape(1, 128), W2.reshape(1, 128), att_src2.reshape(1, 1))


def _sc_d(T, c2v, src_r, dst_r, zer1, NP, nchunks):
    mesh = plsc.VectorSubcoreMesh(core_axis_name="c", subcore_axis_name="s",
                                  num_cores=NC, num_subcores=NS)
    tecrows = NP // NS

    @functools.partial(
        pl.kernel,
        out_type=jax.ShapeDtypeStruct((NC * 2 * NP,), jnp.float32),
        mesh=mesh,
        scratch_types=[
            pltpu.VMEM((NP,), jnp.float32),
            pltpu.VMEM((NP,), jnp.float32),
            pltpu.VMEM((16,), jnp.float32),
            pltpu.VMEM((K,), jnp.int32),
            pltpu.VMEM((K,), jnp.int32),
            pltpu.VMEM((K,), jnp.float32),
            pltpu.VMEM((K,), jnp.float32),
            pltpu.VMEM((NP // NS,), jnp.float32),
            pltpu.VMEM_SHARED((NP,), jnp.float32),
            pltpu.VMEM_SHARED((NP,), jnp.float32),
        ],
        compiler_params=pltpu.CompilerParams(needs_layout_passes=False, use_tc_tiling_on_sc=False),
    )
    def kern(t_h, c2_h, src_h, dst_h, z1_h, ndp,
             za_t, z_t, c2b, srcb, dstb, nbuf, wbuf, zbuf, accn, accd):
        c = lax.axis_index("c")
        s = lax.axis_index("s")
        wid = s * NC + c
        row0 = s * tecrows
        pltpu.sync_copy(z1_h.at[pl.ds(row0, tecrows)], zbuf)
        pltpu.sync_copy(zbuf, accn.at[pl.ds(row0, tecrows)])
        pltpu.sync_copy(zbuf, accd.at[pl.ds(row0, tecrows)])
        pltpu.sync_copy(t_h.at[0], za_t)
        pltpu.sync_copy(t_h.at[1], z_t)
        pltpu.sync_copy(c2_h, c2b)
        plsc.subcore_barrier()
        c2vec = c2b[...]

        @pl.loop(0, nchunks)
        def _chunk(ci):
            pltpu.sync_copy(src_h.at[wid, ci], srcb)
            pltpu.sync_copy(dst_h.at[wid, ci], dstb)
            for g in range(K // 16):
                s16 = srcb[pl.ds(g * 16, 16)]
                d16 = dstb[pl.ds(g * 16, 16)]
                dg = jnp.minimum(d16, NP - 1)
                za = plsc.load_gather(za_t, [s16])
                zs = plsc.load_gather(z_t, [s16])
                zd = plsc.load_gather(z_t, [dg])
                e = za + zd * c2vec
                e = jnp.where(e > 0, e, 0.2 * e)
                w2 = jnp.exp(e)
                nbuf[pl.ds(g * 16, 16)] = w2 * zs
                wbuf[pl.ds(g * 16, 16)] = w2
            pltpu.sync_copy(nbuf, accn.at[dstb], add=True)
            pltpu.sync_copy(wbuf, accd.at[dstb], add=True)

        plsc.subcore_barrier()
        off = c * 2 * NP + s * tecrows
        pltpu.sync_copy(accn.at[pl.ds(s * tecrows, tecrows)], zbuf)
        pltpu.sync_copy(zbuf, ndp.at[pl.ds(off, tecrows)])
        pltpu.sync_copy(accd.at[pl.ds(s * tecrows, tecrows)], zbuf)
        pltpu.sync_copy(zbuf, ndp.at[pl.ds(off + NP, tecrows)])

    return kern(T, c2v, src_r, dst_r, zer1)


def _tc_e_body(nd_ref, b2_ref, out_ref):
    num = nd_ref[0, 0, :] + nd_ref[1, 0, :]
    den = nd_ref[0, 1, :] + nd_ref[1, 1, :]
    out_ref[...] = (num / (den + 1e-16) + b2_ref[0, 0])[:, None]


def _tc_e(ndp, b2, NP):
    grid = NP // BN
    nd4 = ndp.reshape(NC, 2, NP)
    return pl.pallas_call(
        _tc_e_body,
        grid=(grid,),
        in_specs=[
            pl.BlockSpec((NC, 2, BN), lambda i: (0, 0, i)),
            pl.BlockSpec((1, 1), lambda i: (0, 0)),
        ],
        out_specs=pl.BlockSpec((BN, 1), lambda i: (i, 0)),
        out_shape=jax.ShapeDtypeStruct((NP, 1), jnp.float32),
    )(nd4, b2.reshape(1, 1))


def kernel(x, edge_index, W1, b1, att_src1, att_dst1, W2, b2, att_src2, att_dst2):
    N = x.shape[0]
    E = edge_index.shape[1]
    NP = _cdiv(N, BN) * BN
    while NP % (NS * 8) != 0:
        NP += BN
    nchunks = _cdiv(_cdiv(E, NW), K)
    Epad = NW * nchunks * K

    x_p = jnp.pad(x, ((0, NP - N), (0, 0)))
    src = edge_index[0].astype(jnp.int32)
    dst = edge_index[1].astype(jnp.int32)
    src_r = jnp.pad(src, (0, Epad - E)).reshape(NW, nchunks, K)
    dst_r = jnp.pad(dst, (0, Epad - E),
                    constant_values=N).reshape(NW, nchunks, K)
    zer32 = jnp.zeros((NP, 32), jnp.float32)
    zer1 = jnp.zeros((NP,), jnp.float32)

    *hh, att = _tc_a(x_p, W1, att_src1, att_dst1, NP)
    aggp, denp, _wr = _sc_b(tuple(hh), att, src_r, dst_r, zer32, zer1,
                            NP, nchunks)
    T = _tc_c(aggp, denp, b1, W2, att_src2, NP)
    c2v = jnp.full((16,), att_dst2[0, 0], jnp.float32)
    ndp = _sc_d(T, c2v, src_r, dst_r, zer1, NP, nchunks)
    out = _tc_e(ndp, b2, NP)
    return out[:N]
